# baseline (speedup 1.0000x reference)
import math

import jax
import jax.numpy as jnp
from jax.experimental import pallas as pl
from jax.experimental.pallas import tpu as pltpu

_HID = 64
_SPECS = ((4, 64), (8, 32), (16, 16))
_BLK = 2048
_CHUNK = 256


def _fused_body(q_ref, m0_ref, m1_ref, m2_ref,
                wq0_ref, bq0_ref, sg0_ref, st0_ref, u0_ref, vb0_ref,
                wq1_ref, bq1_ref, sg1_ref, st1_ref, u1_ref, vb1_ref,
                wq2_ref, bq2_ref, sg2_ref, st2_ref, u2_ref, vb2_ref,
                bc_ref, o_ref):
    tiers = (
        (m0_ref, wq0_ref, bq0_ref, sg0_ref, st0_ref, u0_ref, vb0_ref),
        (m1_ref, wq1_ref, bq1_ref, sg1_ref, st1_ref, u1_ref, vb1_ref),
        (m2_ref, wq2_ref, bq2_ref, sg2_ref, st2_ref, u2_ref, vb2_ref),
    )
    bc = bc_ref[...]
    nchunk = _BLK // _CHUNK
    for c in range(nchunk):
        sl = slice(c * _CHUNK, (c + 1) * _CHUNK)
        q = q_ref[sl, :]
        outs, confs, vbs = [], [], []
        for (m_ref, wq_ref, bq_ref, sg_ref, st_ref, u_ref, vb_ref) in tiers:
            m = m_ref[sl, :]
            qt = jnp.dot(q, wq_ref[...],
                         preferred_element_type=jnp.float32) + bq_ref[...]
            sims = jnp.dot(m * qt, sg_ref[...],
                           preferred_element_type=jnp.float32)
            mx = jnp.max(sims, axis=-1, keepdims=True)
            e = jnp.exp(sims - mx)
            ssum = jnp.sum(e, axis=-1, keepdims=True)
            conf = 1.0 / ssum
            wexp = jnp.dot(e, st_ref[...],
                           preferred_element_type=jnp.float32)
            o = jnp.dot(wexp * m, u_ref[...],
                        preferred_element_type=jnp.float32)
            outs.append(o)
            confs.append(conf)
            vbs.append(vb_ref[...])
        cmx = jnp.maximum(jnp.maximum(confs[0], confs[1]), confs[2])
        eg = [jnp.exp(cf - cmx) for cf in confs]
        ginv = 1.0 / (eg[0] + eg[1] + eg[2])
        acc = bc
        for i in range(3):
            gi = eg[i] * ginv
            acc = acc + (gi * confs[i]) * outs[i] + gi * vbs[i]
        o_ref[sl, :] = acc


def kernel(query_h, mem0, mem1, mem2, Wp0, bp0, Wp1, bp1, Wp2, bp2,
           Wu0, bu0, Wu1, bu1, Wu2, bu2, Wc, bc):
    B = query_h.shape[0]
    hp = jax.lax.Precision.HIGHEST
    mems = (mem0.reshape(B, -1), mem1.reshape(B, -1), mem2.reshape(B, -1))
    Wps, bps = (Wp0, Wp1, Wp2), (bp0, bp1, bp2)
    Wus, bus = (Wu0, Wu1, Wu2), (bu0, bu1, bu2)

    weight_args = []
    weight_specs = []

    def _w(arr):
        arr = arr.astype(jnp.float32)
        weight_args.append(arr)
        weight_specs.append(
            pl.BlockSpec(arr.shape, lambda i: (0,) * arr.ndim))

    for i, (S, d) in enumerate(_SPECS):
        scale = 1.0 / math.sqrt(d)
        _w(jnp.tile(Wps[i].T * scale, (1, S)))
        _w(jnp.tile(bps[i] * scale, S).reshape(1, S * d))
        seg = jnp.repeat(jnp.eye(S, dtype=jnp.float32), d, axis=0)
        _w(seg)
        _w(seg.T)
        _w(jnp.tile(jnp.dot(Wus[i].T, Wc.T, precision=hp), (S, 1)))
        _w(jnp.dot(bus[i], Wc.T, precision=hp).reshape(1, _HID))
    _w(bc.reshape(1, _HID))

    grid = (B // _BLK,)
    data_specs = [
        pl.BlockSpec((_BLK, _HID), lambda i: (i, 0)),
        pl.BlockSpec((_BLK, 256), lambda i: (i, 0)),
        pl.BlockSpec((_BLK, 256), lambda i: (i, 0)),
        pl.BlockSpec((_BLK, 256), lambda i: (i, 0)),
    ]
    out = pl.pallas_call(
        _fused_body,
        out_shape=jax.ShapeDtypeStruct((B, _HID), jnp.float32),
        grid=grid,
        in_specs=data_specs + weight_specs,
        out_specs=pl.BlockSpec((_BLK, _HID), lambda i: (i, 0)),
        compiler_params=pltpu.CompilerParams(
            dimension_semantics=("parallel",),
            vmem_limit_bytes=48 * 1024 * 1024,
        ),
        name="simultaneous_retrieval_fused",
    )(query_h, *mems, *weight_args)
    return out


# speedup vs baseline: 1.3042x; 1.3042x over previous
import math

import jax
import jax.numpy as jnp
from jax.experimental import pallas as pl
from jax.experimental.pallas import tpu as pltpu

_HID = 64
_SPECS = ((4, 64), (8, 32), (16, 16))
_BLK = 2048
_CHUNK = 512


def _fused_body(q_ref, m0_ref, m1_ref, m2_ref,
                wq0_ref, bq0_ref, sg0_ref, st0_ref, u0_ref, vb0_ref,
                wq1_ref, bq1_ref, sg1_ref, st1_ref, u1_ref, vb1_ref,
                wq2_ref, bq2_ref, sg2_ref, st2_ref, u2_ref, vb2_ref,
                bc_ref, o_ref):
    tiers = (
        (m0_ref, wq0_ref, bq0_ref, sg0_ref, st0_ref, u0_ref, vb0_ref),
        (m1_ref, wq1_ref, bq1_ref, sg1_ref, st1_ref, u1_ref, vb1_ref),
        (m2_ref, wq2_ref, bq2_ref, sg2_ref, st2_ref, u2_ref, vb2_ref),
    )
    bc = bc_ref[...]
    nchunk = _BLK // _CHUNK
    for c in range(nchunk):
        sl = slice(c * _CHUNK, (c + 1) * _CHUNK)
        q = q_ref[sl, :]
        outs, confs, vbs = [], [], []
        for (m_ref, wq_ref, bq_ref, sg_ref, st_ref, u_ref, vb_ref) in tiers:
            m = m_ref[sl, :]
            qt = jnp.dot(q, wq_ref[...],
                         preferred_element_type=jnp.float32) + bq_ref[...]
            sims = jnp.dot(m * qt, sg_ref[...],
                           preferred_element_type=jnp.float32)
            mx = jnp.max(sims, axis=-1, keepdims=True)
            e = jnp.exp(sims - mx)
            ssum = jnp.sum(e, axis=-1, keepdims=True)
            conf = 1.0 / ssum
            wexp = jnp.dot(e, st_ref[...],
                           preferred_element_type=jnp.float32)
            o = jnp.dot(wexp * m, u_ref[...],
                        preferred_element_type=jnp.float32)
            outs.append(o)
            confs.append(conf)
            vbs.append(vb_ref[...])
        cmx = jnp.maximum(jnp.maximum(confs[0], confs[1]), confs[2])
        eg = [jnp.exp(cf - cmx) for cf in confs]
        ginv = 1.0 / (eg[0] + eg[1] + eg[2])
        acc = bc
        for i in range(3):
            gi = eg[i] * ginv
            acc = acc + (gi * confs[i]) * outs[i] + gi * vbs[i]
        o_ref[sl, :] = acc


def kernel(query_h, mem0, mem1, mem2, Wp0, bp0, Wp1, bp1, Wp2, bp2,
           Wu0, bu0, Wu1, bu1, Wu2, bu2, Wc, bc):
    B = query_h.shape[0]
    hp = jax.lax.Precision.HIGHEST
    mems = (mem0.reshape(B, -1), mem1.reshape(B, -1), mem2.reshape(B, -1))
    Wps, bps = (Wp0, Wp1, Wp2), (bp0, bp1, bp2)
    Wus, bus = (Wu0, Wu1, Wu2), (bu0, bu1, bu2)

    weight_args = []
    weight_specs = []

    def _w(arr):
        arr = arr.astype(jnp.float32)
        weight_args.append(arr)
        weight_specs.append(
            pl.BlockSpec(arr.shape, lambda i: (0,) * arr.ndim))

    for i, (S, d) in enumerate(_SPECS):
        scale = 1.0 / math.sqrt(d)
        _w(jnp.tile(Wps[i].T * scale, (1, S)))
        _w(jnp.tile(bps[i] * scale, S).reshape(1, S * d))
        seg = jnp.repeat(jnp.eye(S, dtype=jnp.float32), d, axis=0)
        _w(seg)
        _w(seg.T)
        _w(jnp.tile(jnp.dot(Wus[i].T, Wc.T, precision=hp), (S, 1)))
        _w(jnp.dot(bus[i], Wc.T, precision=hp).reshape(1, _HID))
    _w(bc.reshape(1, _HID))

    grid = (B // _BLK,)
    data_specs = [
        pl.BlockSpec((_BLK, _HID), lambda i: (i, 0)),
        pl.BlockSpec((_BLK, 256), lambda i: (i, 0)),
        pl.BlockSpec((_BLK, 256), lambda i: (i, 0)),
        pl.BlockSpec((_BLK, 256), lambda i: (i, 0)),
    ]
    out = pl.pallas_call(
        _fused_body,
        out_shape=jax.ShapeDtypeStruct((B, _HID), jnp.float32),
        grid=grid,
        in_specs=data_specs + weight_specs,
        out_specs=pl.BlockSpec((_BLK, _HID), lambda i: (i, 0)),
        compiler_params=pltpu.CompilerParams(
            dimension_semantics=("parallel",),
            vmem_limit_bytes=48 * 1024 * 1024,
        ),
        name="simultaneous_retrieval_fused",
    )(query_h, *mems, *weight_args)
    return out


# speedup vs baseline: 1.3593x; 1.0422x over previous
import math

import jax
import jax.numpy as jnp
from jax.experimental import pallas as pl
from jax.experimental.pallas import tpu as pltpu

_HID = 64
_SPECS = ((4, 64), (8, 32), (16, 16))
_BLK = 2048
_CHUNK = 512
_NEG = -1e30


def _fused_body(q_ref, m0_ref, m1_ref, m2_ref,
                wq_ref, bq_ref, seg_ref, msk_ref,
                st0_ref, st1_ref, st2_ref, u_ref,
                vb0_ref, vb1_ref, vb2_ref, bc_ref, o_ref):
    bc = bc_ref[...]
    vbs = (vb0_ref[...], vb1_ref[...], vb2_ref[...])
    sts = (st0_ref[...], st1_ref[...], st2_ref[...])
    for c in range(_BLK // _CHUNK):
        sl = slice(c * _CHUNK, (c + 1) * _CHUNK)
        q = q_ref[sl, :]
        ms = (m0_ref[sl, :], m1_ref[sl, :], m2_ref[sl, :])
        m_all = jnp.concatenate(ms, axis=1)
        qt = jnp.dot(q, wq_ref[...],
                     preferred_element_type=jnp.float32) + bq_ref[...]
        sims = jnp.dot(m_all * qt, seg_ref[...],
                       preferred_element_type=jnp.float32) + msk_ref[...]
        es, confs = [], []
        for t in range(3):
            st = sims[:, 128 * t:128 * (t + 1)]
            mx = jnp.max(st, axis=-1, keepdims=True)
            e = jnp.exp(st - mx)
            ssum = jnp.sum(e, axis=-1, keepdims=True)
            es.append(e)
            confs.append(1.0 / ssum)
        cmx = jnp.maximum(jnp.maximum(confs[0], confs[1]), confs[2])
        eg = [jnp.exp(cf - cmx) for cf in confs]
        ginv = 1.0 / (eg[0] + eg[1] + eg[2])
        acc = bc
        pms = []
        for t in range(3):
            gt = eg[t] * ginv
            acc = acc + gt * vbs[t]
            wexp = jnp.dot((gt * confs[t]) * es[t], sts[t],
                           preferred_element_type=jnp.float32)
            pms.append(wexp * ms[t])
        out = jnp.dot(jnp.concatenate(pms, axis=1), u_ref[...],
                      preferred_element_type=jnp.float32)
        o_ref[sl, :] = out + acc


def kernel(query_h, mem0, mem1, mem2, Wp0, bp0, Wp1, bp1, Wp2, bp2,
           Wu0, bu0, Wu1, bu1, Wu2, bu2, Wc, bc):
    B = query_h.shape[0]
    hp = jax.lax.Precision.HIGHEST
    mems = (mem0.reshape(B, -1), mem1.reshape(B, -1), mem2.reshape(B, -1))
    Wps, bps = (Wp0, Wp1, Wp2), (bp0, bp1, bp2)
    Wus, bus = (Wu0, Wu1, Wu2), (bu0, bu1, bu2)

    wq_cols, bq_cols, u_rows = [], [], []
    seg = jnp.zeros((768, 384), dtype=jnp.float32)
    msk = jnp.full((1, 384), _NEG, dtype=jnp.float32)
    sts, vbs = [], []
    off = 0
    for i, (S, d) in enumerate(_SPECS):
        scale = 1.0 / math.sqrt(d)
        wq_cols.append(jnp.tile(Wps[i].T * scale, (1, S)))
        bq_cols.append(jnp.tile(bps[i] * scale, S))
        eye = jnp.eye(S, dtype=jnp.float32)
        seg_i = jnp.repeat(eye, d, axis=0)
        seg = seg.at[off:off + 256, 128 * i:128 * i + S].set(seg_i)
        msk = msk.at[0, 128 * i:128 * i + S].set(0.0)
        st_i = jnp.zeros((128, 256), dtype=jnp.float32)
        st_i = st_i.at[:S, :].set(seg_i.T)
        sts.append(st_i)
        u_rows.append(jnp.tile(jnp.dot(Wus[i].T, Wc.T, precision=hp), (S, 1)))
        vbs.append(jnp.dot(bus[i], Wc.T, precision=hp).reshape(1, _HID))
        off += 256
    wq = jnp.concatenate(wq_cols, axis=1)
    bq = jnp.concatenate(bq_cols).reshape(1, 768)
    u_all = jnp.concatenate(u_rows, axis=0)
    bc2 = bc.reshape(1, _HID)

    weight_args = [wq, bq, seg, msk, *sts, u_all, *vbs, bc2]
    weight_specs = [
        pl.BlockSpec(a.shape, lambda i: (0,) * a.ndim) for a in weight_args
    ]

    grid = (B // _BLK,)
    data_specs = [
        pl.BlockSpec((_BLK, _HID), lambda i: (i, 0)),
        pl.BlockSpec((_BLK, 256), lambda i: (i, 0)),
        pl.BlockSpec((_BLK, 256), lambda i: (i, 0)),
        pl.BlockSpec((_BLK, 256), lambda i: (i, 0)),
    ]
    out = pl.pallas_call(
        _fused_body,
        out_shape=jax.ShapeDtypeStruct((B, _HID), jnp.float32),
        grid=grid,
        in_specs=data_specs + weight_specs,
        out_specs=pl.BlockSpec((_BLK, _HID), lambda i: (i, 0)),
        compiler_params=pltpu.CompilerParams(
            dimension_semantics=("parallel",),
            vmem_limit_bytes=48 * 1024 * 1024,
        ),
        name="simultaneous_retrieval_fused",
    )(query_h, *mems, *weight_args)
    return out
